# baseline (speedup 1.0000x reference)
import functools

import jax
import jax.numpy as jnp
from jax import lax
from jax.experimental import pallas as pl
from jax.experimental.pallas import tpu as pltpu
from jax.experimental.pallas import tpu_sc as plsc

_B, _D = 16384, 64
_NW = 32
_RPW = _B // _NW
_NG = _RPW // 16
_EPS = 1e-12


def _sc_body(guf, gif, gisf, xui_h, guo, gio, giso,
             gu_v, gi_v, gis_v, xui_v, sem_in, sem_out):
    wid = lax.axis_index("s") * 2 + lax.axis_index("c")
    rbase = wid * _RPW

    for j in range(_D):
        src = pl.ds(j * _B + rbase, _RPW)
        dst = pl.ds(j * _RPW, _RPW)
        pltpu.async_copy(guf.at[src], gu_v.at[dst], sem_in)
        pltpu.async_copy(gif.at[src], gi_v.at[dst], sem_in)
        pltpu.async_copy(gisf.at[src], gis_v.at[dst], sem_in)
    pltpu.make_async_copy(guf.at[pl.ds(0, _RPW * _D)], gu_v, sem_in).wait()
    pltpu.make_async_copy(guf.at[pl.ds(0, _RPW * _D)], gi_v, sem_in).wait()
    pltpu.make_async_copy(guf.at[pl.ds(0, _RPW * _D)], gis_v, sem_in).wait()

    for j in range(_D):
        src = pl.ds(j * _RPW, _RPW)
        dst = pl.ds(j * _B + rbase, _RPW)
        pltpu.async_copy(gu_v.at[src], guo.at[dst], sem_out)
        pltpu.async_copy(gi_v.at[src], gio.at[dst], sem_out)
        pltpu.async_copy(gis_v.at[src], giso.at[dst], sem_out)

    def group(g, carry):
        a = jnp.zeros((16,), jnp.float32)
        b = jnp.zeros((16,), jnp.float32)
        c = jnp.zeros((16,), jnp.float32)
        for j in range(_D):
            sl = pl.ds(j * _RPW + g * 16, 16)
            u = gu_v[sl]
            i_ = gi_v[sl]
            s = gis_v[sl]
            a = a + u * i_
            b = b + u * s
            c = c + s * s
        cc = jnp.maximum(c, 1e-30)
        y = plsc.bitcast(0x5F3759DF - (plsc.bitcast(cc, jnp.int32) >> 1),
                         jnp.float32)
        y = y * (1.5 - 0.5 * cc * y * y)
        y = y * (1.5 - 0.5 * cc * y * y)
        y = y * (1.5 - 0.5 * cc * y * y)
        d = jnp.maximum(cc * y, _EPS)
        xui_v[pl.ds(g * 16, 16)] = a + b / d
        return carry

    lax.fori_loop(0, _NG, group, 0)
    pltpu.sync_copy(xui_v, xui_h.at[pl.ds(rbase, _RPW)])
    pltpu.make_async_copy(guf.at[pl.ds(0, _RPW * _D)], gu_v, sem_out).wait()
    pltpu.make_async_copy(guf.at[pl.ds(0, _RPW * _D)], gi_v, sem_out).wait()
    pltpu.make_async_copy(guf.at[pl.ds(0, _RPW * _D)], gis_v, sem_out).wait()


_mesh = plsc.VectorSubcoreMesh(core_axis_name="c", subcore_axis_name="s")

_sc_kernel = functools.partial(
    pl.kernel,
    out_type=(
        jax.ShapeDtypeStruct((_B,), jnp.float32),
        jax.ShapeDtypeStruct((_B * _D,), jnp.float32),
        jax.ShapeDtypeStruct((_B * _D,), jnp.float32),
        jax.ShapeDtypeStruct((_B * _D,), jnp.float32),
    ),
    mesh=_mesh,
    compiler_params=pltpu.CompilerParams(needs_layout_passes=False),
    scratch_types=[
        pltpu.VMEM((_RPW * _D,), jnp.float32),
        pltpu.VMEM((_RPW * _D,), jnp.float32),
        pltpu.VMEM((_RPW * _D,), jnp.float32),
        pltpu.VMEM((_RPW,), jnp.float32),
        pltpu.SemaphoreType.DMA,
        pltpu.SemaphoreType.DMA,
    ],
)(_sc_body)


def kernel(gu, gi, gis):
    xui, guoT, gioT, gisoT = _sc_kernel(
        gu.T.reshape(-1), gi.T.reshape(-1), gis.T.reshape(-1))
    return (xui,
            guoT.reshape(_D, _B).T,
            gioT.reshape(_D, _B).T,
            gisoT.reshape(_D, _B).T)


# speedup vs baseline: 6.2041x; 6.2041x over previous
import jax
import jax.numpy as jnp
from jax.experimental import pallas as pl

_B, _D = 16384, 64
_BLK = 8192
_EPS = 1e-12


def _body(gu_ref, gi_ref, gis_ref, xui_ref, guo_ref, gio_ref, giso_ref):
    gu = gu_ref[...]
    gi = gi_ref[...]
    gis = gis_ref[...]
    guo_ref[...] = gu
    gio_ref[...] = gi
    giso_ref[...] = gis
    c = jnp.sum(gis * gis, axis=0)
    inv = 1.0 / jnp.maximum(jnp.sqrt(c), _EPS)
    f = gi + gis * inv[None, :]
    xui_ref[...] = jnp.sum(gu * f, axis=0)


def kernel(gu, gi, gis):
    guT = gu.T
    giT = gi.T
    gisT = gis.T
    col = pl.BlockSpec((_D, _BLK), lambda i: (0, i))
    xui, guoT, gioT, gisoT = pl.pallas_call(
        _body,
        grid=(_B // _BLK,),
        in_specs=[col, col, col],
        out_specs=(pl.BlockSpec((_BLK,), lambda i: (i,)), col, col, col),
        out_shape=(
            jax.ShapeDtypeStruct((_B,), jnp.float32),
            jax.ShapeDtypeStruct((_D, _B), jnp.float32),
            jax.ShapeDtypeStruct((_D, _B), jnp.float32),
            jax.ShapeDtypeStruct((_D, _B), jnp.float32),
        ),
    )(guT, giT, gisT)
    return (xui, guoT.T, gioT.T, gisoT.T)
